# baseline (speedup 1.0000x reference)
import functools

import jax
import jax.numpy as jnp
from jax import lax
from jax.experimental import pallas as pl
from jax.experimental.pallas import tpu as pltpu
from jax.experimental.pallas import tpu_sc as plsc

_NB1 = 1024
_NBR = 2048
_NC, _NS, _L = 2, 16, 16
_NW = _NC * _NS
_CH = 64


def _k1nn(x2t_ref, x1_ref, gidx_ref, w_ref, *, M):
    b = pl.program_id(0)
    nb = x1_ref.shape[1]

    x2 = x2t_ref[...]
    x1 = x1_ref[...]
    n2 = jnp.sum(x2 * x2, axis=1, keepdims=True)
    n1 = jnp.sum(x1 * x1, axis=0, keepdims=True)
    x2a = jnp.concatenate([x2, n2, jnp.ones_like(n2)], axis=1)
    x1a = jnp.concatenate([-2.0 * x1, jnp.ones_like(n1), n1], axis=0)
    d = jax.lax.dot_general(
        x2a,
        x1a,
        (((1,), (0,)), ((), ())),
        precision=jax.lax.Precision.HIGHEST,
        preferred_element_type=jnp.float32,
    )

    inf = jnp.float32(jnp.inf)
    sub_iota = jax.lax.broadcasted_iota(jnp.int32, (M, nb), 0)
    m1 = jnp.min(d, axis=0, keepdims=True)
    m2 = jnp.min(jnp.where(d > m1, d, inf), axis=0, keepdims=True)
    m3 = jnp.min(jnp.where(d > m2, d, inf), axis=0, keepdims=True)
    i1 = jnp.min(jnp.where(d == m1, sub_iota, M), axis=0, keepdims=True)
    i2 = jnp.min(jnp.where(d == m2, sub_iota, M), axis=0, keepdims=True)
    i3 = jnp.min(jnp.where(d == m3, sub_iota, M), axis=0, keepdims=True)

    r1 = 1.0 / jnp.maximum(m1, 1e-10)
    r2 = 1.0 / jnp.maximum(m2, 1e-10)
    r3 = 1.0 / jnp.maximum(m3, 1e-10)
    rs = r1 + r2 + r3
    gidx_ref[...] = jnp.concatenate([i1, i2, i3], axis=0) + b * M
    w_ref[...] = jnp.stack(
        [
            jnp.broadcast_to((r1 / rs)[0][:, None], (nb, 16)),
            jnp.broadcast_to((r2 / rs)[0][:, None], (nb, 16)),
            jnp.broadcast_to((r3 / rs)[0][:, None], (nb, 16)),
        ],
        axis=0,
    )


def _make_sc_interp(P, C2):
    PW = P // _NW
    nch = PW // _CH
    mesh = plsc.VectorSubcoreMesh(core_axis_name="c", subcore_axis_name="s")

    @functools.partial(
        pl.kernel,
        mesh=mesh,
        out_type=jax.ShapeDtypeStruct((P, C2), jnp.float32),
        scratch_types=[
            pltpu.VMEM((3, _CH), jnp.int32),
            pltpu.VMEM((3 * _CH * _L,), jnp.float32),
            pltpu.VMEM((_CH, C2), jnp.float32),
            pltpu.VMEM((_CH, C2), jnp.float32),
            pltpu.VMEM((_CH, C2), jnp.float32),
            pltpu.VMEM((_CH, C2), jnp.float32),
            pltpu.SemaphoreType.DMA,
        ],
    )
    def k(table_hbm, gidx_hbm, w_hbm, out_hbm, idx_v, w_v, ra_v, rb_v, rc_v, acc_v, sem):
        wid = lax.axis_index("s") * _NC + lax.axis_index("c")
        base0 = wid * PW
        rows = (ra_v, rb_v, rc_v)

        def chunk_body(ci, carry):
            base = base0 + ci * _CH
            for k3 in range(3):
                pltpu.sync_copy(gidx_hbm.at[pl.ds(k3 * P + base, _CH)], idx_v.at[k3])
                pltpu.sync_copy(
                    w_hbm.at[pl.ds((k3 * P + base) * _L, _CH * _L)],
                    w_v.at[pl.ds(k3 * _CH * _L, _CH * _L)],
                )
            for k3 in range(3):
                pltpu.async_copy(table_hbm.at[idx_v.at[k3]], rows[k3], sem)
            for k3 in range(3):
                pltpu.make_async_copy(table_hbm.at[idx_v.at[k3]], rows[k3], sem).wait()

            def pt_body(p, c2):
                w0 = w_v[pl.ds((0 * _CH + p) * _L, _L)]
                w1 = w_v[pl.ds((1 * _CH + p) * _L, _L)]
                w2 = w_v[pl.ds((2 * _CH + p) * _L, _L)]
                for j in range(C2 // _L):
                    sl = pl.ds(j * _L, _L)
                    acc_v[p, sl] = (
                        w0 * ra_v[p, sl] + w1 * rb_v[p, sl] + w2 * rc_v[p, sl]
                    )
                return c2

            lax.fori_loop(0, _CH, pt_body, 0, unroll=False)
            pltpu.sync_copy(acc_v, out_hbm.at[pl.ds(base, _CH)])
            return carry

        lax.fori_loop(0, nch, chunk_body, 0, unroll=False)

    return k


def _k3row(int_ref, p1_ref, w1a_ref, w1b_ref, h_ref, s_ref, ss_ref):
    t = pl.program_id(0)
    h = jnp.dot(int_ref[...], w1a_ref[...], preferred_element_type=jnp.float32)
    h = h + jnp.dot(p1_ref[...], w1b_ref[...], preferred_element_type=jnp.float32)
    h_ref[...] = h

    @pl.when(t == 0)
    def _init():
        s_ref[...] = jnp.zeros_like(s_ref)
        ss_ref[...] = jnp.zeros_like(ss_ref)

    s_ref[...] += jnp.sum(h, axis=0, keepdims=True)
    ss_ref[...] += jnp.sum(h * h, axis=0, keepdims=True)


def _bn_ac_row(s_ref, ss_ref, g_ref, be_ref, cnt):
    mean = s_ref[...] * (1.0 / cnt)
    var = jnp.maximum(ss_ref[...] * (1.0 / cnt) - mean * mean, 0.0)
    a = g_ref[...] * jax.lax.rsqrt(var + 1e-5)
    c = be_ref[...] - mean * a
    return a, c


def _k4row(h_ref, sin_ref, ssin_ref, g_ref, be_ref, w2_ref, h2_ref, s_ref, ss_ref, *, cnt):
    t = pl.program_id(0)
    a, c = _bn_ac_row(sin_ref, ssin_ref, g_ref, be_ref, cnt)
    hn = jnp.maximum(a * h_ref[...] + c, 0.0)
    h2 = jnp.dot(hn, w2_ref[...], preferred_element_type=jnp.float32)
    h2_ref[...] = h2

    @pl.when(t == 0)
    def _init():
        s_ref[...] = jnp.zeros_like(s_ref)
        ss_ref[...] = jnp.zeros_like(ss_ref)

    s_ref[...] += jnp.sum(h2, axis=0, keepdims=True)
    ss_ref[...] += jnp.sum(h2 * h2, axis=0, keepdims=True)


def _k5row(h2_ref, sin_ref, ssin_ref, g_ref, be_ref, o_ref, *, cnt):
    a, c = _bn_ac_row(sin_ref, ssin_ref, g_ref, be_ref, cnt)
    o_ref[...] = jnp.maximum(a * h2_ref[...] + c, 0.0)


def kernel(xyz1, xyz2, points1, points2, W1, b1, g1, be1, W2, b2, g2, be2):
    B, _, N = xyz1.shape
    M = xyz2.shape[2]
    C1 = points1.shape[1]
    C2 = points2.shape[1]
    H1 = W1.shape[0]
    H2 = W2.shape[0]
    P = B * N
    nb1 = min(_NB1, N)
    nbr = min(_NBR, P)
    nt1 = N // nb1

    x2t = jnp.transpose(xyz2, (0, 2, 1))

    gidx, w3 = pl.pallas_call(
        functools.partial(_k1nn, M=M),
        grid=(B, nt1),
        in_specs=[
            pl.BlockSpec((None, M, 3), lambda b, n: (b, 0, 0)),
            pl.BlockSpec((None, 3, nb1), lambda b, n: (b, 0, n)),
        ],
        out_specs=[
            pl.BlockSpec((3, nb1), lambda b, n: (0, b * nt1 + n)),
            pl.BlockSpec((3, nb1, 16), lambda b, n: (0, b * nt1 + n, 0)),
        ],
        out_shape=[
            jax.ShapeDtypeStruct((3, P), jnp.int32),
            jax.ShapeDtypeStruct((3, P, 16), jnp.float32),
        ],
    )(x2t, xyz1)

    table = jnp.transpose(points2, (0, 2, 1)).reshape(B * M, C2)
    interp = _make_sc_interp(P, C2)(table, gidx.reshape(-1), w3.reshape(-1))

    p1r = jnp.transpose(points1, (0, 2, 1)).reshape(P, C1)

    h1r, s1, ss1 = pl.pallas_call(
        _k3row,
        grid=(P // nbr,),
        in_specs=[
            pl.BlockSpec((nbr, C2), lambda t: (t, 0)),
            pl.BlockSpec((nbr, C1), lambda t: (t, 0)),
            pl.BlockSpec((C2, H1), lambda t: (0, 0)),
            pl.BlockSpec((C1, H1), lambda t: (0, 0)),
        ],
        out_specs=[
            pl.BlockSpec((nbr, H1), lambda t: (t, 0)),
            pl.BlockSpec((1, H1), lambda t: (0, 0)),
            pl.BlockSpec((1, H1), lambda t: (0, 0)),
        ],
        out_shape=[
            jax.ShapeDtypeStruct((P, H1), jnp.float32),
            jax.ShapeDtypeStruct((1, H1), jnp.float32),
            jax.ShapeDtypeStruct((1, H1), jnp.float32),
        ],
    )(interp, p1r, W1[:, :C2].T, W1[:, C2:].T)

    h2r, s2, ss2 = pl.pallas_call(
        functools.partial(_k4row, cnt=P),
        grid=(P // nbr,),
        in_specs=[
            pl.BlockSpec((nbr, H1), lambda t: (t, 0)),
            pl.BlockSpec((1, H1), lambda t: (0, 0)),
            pl.BlockSpec((1, H1), lambda t: (0, 0)),
            pl.BlockSpec((1, H1), lambda t: (0, 0)),
            pl.BlockSpec((1, H1), lambda t: (0, 0)),
            pl.BlockSpec((H1, H2), lambda t: (0, 0)),
        ],
        out_specs=[
            pl.BlockSpec((nbr, H2), lambda t: (t, 0)),
            pl.BlockSpec((1, H2), lambda t: (0, 0)),
            pl.BlockSpec((1, H2), lambda t: (0, 0)),
        ],
        out_shape=[
            jax.ShapeDtypeStruct((P, H2), jnp.float32),
            jax.ShapeDtypeStruct((1, H2), jnp.float32),
            jax.ShapeDtypeStruct((1, H2), jnp.float32),
        ],
    )(h1r, s1, ss1, g1[None, :], be1[None, :], W2.T)

    outr = pl.pallas_call(
        functools.partial(_k5row, cnt=P),
        grid=(P // nbr,),
        in_specs=[
            pl.BlockSpec((nbr, H2), lambda t: (t, 0)),
            pl.BlockSpec((1, H2), lambda t: (0, 0)),
            pl.BlockSpec((1, H2), lambda t: (0, 0)),
            pl.BlockSpec((1, H2), lambda t: (0, 0)),
            pl.BlockSpec((1, H2), lambda t: (0, 0)),
        ],
        out_specs=pl.BlockSpec((nbr, H2), lambda t: (t, 0)),
        out_shape=jax.ShapeDtypeStruct((P, H2), jnp.float32),
    )(h2r, s2, ss2, g2[None, :], be2[None, :])

    return jnp.transpose(outr.reshape(B, N, H2), (0, 2, 1))


# speedup vs baseline: 2.1202x; 2.1202x over previous
import functools

import jax
import jax.numpy as jnp
from jax.experimental import pallas as pl

_NB1 = 2048
_NB2 = 1024
_NB3 = 2048


def _k1(x2t_ref, x1_ref, p2_ref, p1_ref, w1_ref, h1_ref, s_ref, ss_ref):
    b = pl.program_id(0)
    nt = pl.program_id(1)
    M = x2t_ref.shape[0]
    C2 = p2_ref.shape[0]

    x2 = x2t_ref[...]
    x1 = x1_ref[...]
    n2 = jnp.sum(x2 * x2, axis=1, keepdims=True)
    n1 = jnp.sum(x1 * x1, axis=0, keepdims=True)
    x2a = jnp.concatenate([x2, n2, jnp.ones_like(n2)], axis=1)
    x1a = jnp.concatenate([-2.0 * x1, jnp.ones_like(n1), n1], axis=0)
    d = jax.lax.dot_general(
        x2a,
        x1a,
        (((1,), (0,)), ((), ())),
        precision=jax.lax.Precision.HIGHEST,
        preferred_element_type=jnp.float32,
    )

    inf = jnp.float32(jnp.inf)
    m1 = jnp.min(d, axis=0, keepdims=True)
    m2 = jnp.min(jnp.where(d > m1, d, inf), axis=0, keepdims=True)
    m3 = jnp.min(jnp.where(d > m2, d, inf), axis=0, keepdims=True)

    r1 = 1.0 / jnp.maximum(m1, 1e-10)
    r2 = 1.0 / jnp.maximum(m2, 1e-10)
    r3 = 1.0 / jnp.maximum(m3, 1e-10)
    rs = r1 + r2 + r3
    st = jnp.where(d <= m3, 1.0 / (jnp.maximum(d, 1e-10) * rs), 0.0)

    interp = jnp.dot(p2_ref[...], st, preferred_element_type=jnp.float32)
    h = jnp.dot(w1_ref[:, :C2], interp, preferred_element_type=jnp.float32)
    h = h + jnp.dot(w1_ref[:, C2:], p1_ref[...], preferred_element_type=jnp.float32)
    h1_ref[...] = h

    @pl.when(jnp.logical_and(b == 0, nt == 0))
    def _init():
        s_ref[...] = jnp.zeros_like(s_ref)
        ss_ref[...] = jnp.zeros_like(ss_ref)

    hh = h * h
    nb = h.shape[1]
    s_ref[...] += sum(h[:, j * 128 : (j + 1) * 128] for j in range(nb // 128))
    ss_ref[...] += sum(hh[:, j * 128 : (j + 1) * 128] for j in range(nb // 128))


def _bn_ac(s_ref, ss_ref, g_ref, be_ref, cnt):
    mean = jnp.sum(s_ref[...], axis=1, keepdims=True) * (1.0 / cnt)
    var = jnp.maximum(
        jnp.sum(ss_ref[...], axis=1, keepdims=True) * (1.0 / cnt) - mean * mean,
        0.0,
    )
    a = g_ref[...] * jax.lax.rsqrt(var + 1e-5)
    c = be_ref[...] - mean * a
    return a, c


def _k2(h1_ref, sin_ref, ssin_ref, g_ref, be_ref, w2_ref, h2_ref, s_ref, ss_ref, *, cnt):
    b = pl.program_id(0)
    nt = pl.program_id(1)
    a, c = _bn_ac(sin_ref, ssin_ref, g_ref, be_ref, cnt)
    hn = jnp.maximum(a * h1_ref[...] + c, 0.0)
    h2 = jnp.dot(w2_ref[...], hn, preferred_element_type=jnp.float32)
    h2_ref[...] = h2

    @pl.when(jnp.logical_and(b == 0, nt == 0))
    def _init():
        s_ref[...] = jnp.zeros_like(s_ref)
        ss_ref[...] = jnp.zeros_like(ss_ref)

    hh = h2 * h2
    nb = h2.shape[1]
    s_ref[...] += sum(h2[:, j * 128 : (j + 1) * 128] for j in range(nb // 128))
    ss_ref[...] += sum(hh[:, j * 128 : (j + 1) * 128] for j in range(nb // 128))


def _k3(h2_ref, sin_ref, ssin_ref, g_ref, be_ref, o_ref, *, cnt):
    a, c = _bn_ac(sin_ref, ssin_ref, g_ref, be_ref, cnt)
    o_ref[...] = jnp.maximum(a * h2_ref[...] + c, 0.0)


def kernel(xyz1, xyz2, points1, points2, W1, b1, g1, be1, W2, b2, g2, be2):
    B, _, N = xyz1.shape
    M = xyz2.shape[2]
    C1 = points1.shape[1]
    C2 = points2.shape[1]
    H1 = W1.shape[0]
    H2 = W2.shape[0]
    cnt = B * N
    nb1 = min(_NB1, N)
    nb2 = min(_NB2, N)
    nb3 = min(_NB3, N)

    x2t = jnp.transpose(xyz2, (0, 2, 1))

    h1, s1, ss1 = pl.pallas_call(
        _k1,
        grid=(B, N // nb1),
        in_specs=[
            pl.BlockSpec((None, M, 3), lambda b, n: (b, 0, 0)),
            pl.BlockSpec((None, 3, nb1), lambda b, n: (b, 0, n)),
            pl.BlockSpec((None, C2, M), lambda b, n: (b, 0, 0)),
            pl.BlockSpec((None, C1, nb1), lambda b, n: (b, 0, n)),
            pl.BlockSpec((H1, C2 + C1), lambda b, n: (0, 0)),
        ],
        out_specs=[
            pl.BlockSpec((None, H1, nb1), lambda b, n: (b, 0, n)),
            pl.BlockSpec((H1, 128), lambda b, n: (0, 0)),
            pl.BlockSpec((H1, 128), lambda b, n: (0, 0)),
        ],
        out_shape=[
            jax.ShapeDtypeStruct((B, H1, N), jnp.float32),
            jax.ShapeDtypeStruct((H1, 128), jnp.float32),
            jax.ShapeDtypeStruct((H1, 128), jnp.float32),
        ],
    )(x2t, xyz1, points2, points1, W1)

    h2, s2, ss2 = pl.pallas_call(
        functools.partial(_k2, cnt=cnt),
        grid=(B, N // nb2),
        in_specs=[
            pl.BlockSpec((None, H1, nb2), lambda b, n: (b, 0, n)),
            pl.BlockSpec((H1, 128), lambda b, n: (0, 0)),
            pl.BlockSpec((H1, 128), lambda b, n: (0, 0)),
            pl.BlockSpec((H1, 1), lambda b, n: (0, 0)),
            pl.BlockSpec((H1, 1), lambda b, n: (0, 0)),
            pl.BlockSpec((H2, H1), lambda b, n: (0, 0)),
        ],
        out_specs=[
            pl.BlockSpec((None, H2, nb2), lambda b, n: (b, 0, n)),
            pl.BlockSpec((H2, 128), lambda b, n: (0, 0)),
            pl.BlockSpec((H2, 128), lambda b, n: (0, 0)),
        ],
        out_shape=[
            jax.ShapeDtypeStruct((B, H2, N), jnp.float32),
            jax.ShapeDtypeStruct((H2, 128), jnp.float32),
            jax.ShapeDtypeStruct((H2, 128), jnp.float32),
        ],
    )(h1, s1, ss1, g1[:, None], be1[:, None], W2)

    out = pl.pallas_call(
        functools.partial(_k3, cnt=cnt),
        grid=(B, N // nb3),
        in_specs=[
            pl.BlockSpec((None, H2, nb3), lambda b, n: (b, 0, n)),
            pl.BlockSpec((H2, 128), lambda b, n: (0, 0)),
            pl.BlockSpec((H2, 128), lambda b, n: (0, 0)),
            pl.BlockSpec((H2, 1), lambda b, n: (0, 0)),
            pl.BlockSpec((H2, 1), lambda b, n: (0, 0)),
        ],
        out_specs=pl.BlockSpec((None, H2, nb3), lambda b, n: (b, 0, n)),
        out_shape=jax.ShapeDtypeStruct((B, H2, N), jnp.float32),
    )(h2, s2, ss2, g2[:, None], be2[:, None])

    return out


# speedup vs baseline: 2.2431x; 1.0580x over previous
import functools

import jax
import jax.numpy as jnp
from jax.experimental import pallas as pl

_NB1 = 4096
_NB2 = 2048
_NB3 = 2048


def _k1(x2t_ref, x1_ref, p2_ref, p1_ref, w1_ref, h1_ref, s_ref, ss_ref):
    b = pl.program_id(0)
    nt = pl.program_id(1)
    M = x2t_ref.shape[0]
    C2 = p2_ref.shape[0]

    x2 = x2t_ref[...]
    x1 = x1_ref[...]
    n2 = jnp.sum(x2 * x2, axis=1, keepdims=True)
    n1 = jnp.sum(x1 * x1, axis=0, keepdims=True)
    x2a = jnp.concatenate([x2, n2, jnp.ones_like(n2)], axis=1)
    x1a = jnp.concatenate([-2.0 * x1, jnp.ones_like(n1), n1], axis=0)
    d = jax.lax.dot_general(
        x2a,
        x1a,
        (((1,), (0,)), ((), ())),
        precision=jax.lax.Precision.HIGHEST,
        preferred_element_type=jnp.float32,
    )

    inf = jnp.float32(jnp.inf)
    m1 = jnp.min(d, axis=0, keepdims=True)
    m2 = jnp.min(jnp.where(d > m1, d, inf), axis=0, keepdims=True)
    m3 = jnp.min(jnp.where(d > m2, d, inf), axis=0, keepdims=True)

    r1 = 1.0 / jnp.maximum(m1, 1e-10)
    r2 = 1.0 / jnp.maximum(m2, 1e-10)
    r3 = 1.0 / jnp.maximum(m3, 1e-10)
    rs = r1 + r2 + r3
    st = jnp.where(d <= m3, 1.0 / (jnp.maximum(d, 1e-10) * rs), 0.0)

    interp = jnp.dot(p2_ref[...], st, preferred_element_type=jnp.float32)
    h = jnp.dot(w1_ref[:, :C2], interp, preferred_element_type=jnp.float32)
    h = h + jnp.dot(w1_ref[:, C2:], p1_ref[...], preferred_element_type=jnp.float32)
    h1_ref[...] = h

    @pl.when(jnp.logical_and(b == 0, nt == 0))
    def _init():
        s_ref[...] = jnp.zeros_like(s_ref)
        ss_ref[...] = jnp.zeros_like(ss_ref)

    hh = h * h
    nb = h.shape[1]
    s_ref[...] += sum(h[:, j * 128 : (j + 1) * 128] for j in range(nb // 128))
    ss_ref[...] += sum(hh[:, j * 128 : (j + 1) * 128] for j in range(nb // 128))


def _bn_ac(s_ref, ss_ref, g_ref, be_ref, cnt):
    mean = jnp.sum(s_ref[...], axis=1, keepdims=True) * (1.0 / cnt)
    var = jnp.maximum(
        jnp.sum(ss_ref[...], axis=1, keepdims=True) * (1.0 / cnt) - mean * mean,
        0.0,
    )
    a = g_ref[...] * jax.lax.rsqrt(var + 1e-5)
    c = be_ref[...] - mean * a
    return a, c


def _k2(h1_ref, sin_ref, ssin_ref, g_ref, be_ref, w2_ref, h2_ref, s_ref, ss_ref, *, cnt):
    b = pl.program_id(0)
    nt = pl.program_id(1)
    a, c = _bn_ac(sin_ref, ssin_ref, g_ref, be_ref, cnt)
    hn = jnp.maximum(a * h1_ref[...] + c, 0.0)
    h2 = jnp.dot(w2_ref[...], hn, preferred_element_type=jnp.float32)
    h2_ref[...] = h2

    @pl.when(jnp.logical_and(b == 0, nt == 0))
    def _init():
        s_ref[...] = jnp.zeros_like(s_ref)
        ss_ref[...] = jnp.zeros_like(ss_ref)

    hh = h2 * h2
    nb = h2.shape[1]
    s_ref[...] += sum(h2[:, j * 128 : (j + 1) * 128] for j in range(nb // 128))
    ss_ref[...] += sum(hh[:, j * 128 : (j + 1) * 128] for j in range(nb // 128))


def _k3(h2_ref, sin_ref, ssin_ref, g_ref, be_ref, o_ref, *, cnt):
    a, c = _bn_ac(sin_ref, ssin_ref, g_ref, be_ref, cnt)
    o_ref[...] = jnp.maximum(a * h2_ref[...] + c, 0.0)


def kernel(xyz1, xyz2, points1, points2, W1, b1, g1, be1, W2, b2, g2, be2):
    B, _, N = xyz1.shape
    M = xyz2.shape[2]
    C1 = points1.shape[1]
    C2 = points2.shape[1]
    H1 = W1.shape[0]
    H2 = W2.shape[0]
    cnt = B * N
    nb1 = min(_NB1, N)
    nb2 = min(_NB2, N)
    nb3 = min(_NB3, N)

    x2t = jnp.transpose(xyz2, (0, 2, 1))

    h1, s1, ss1 = pl.pallas_call(
        _k1,
        grid=(B, N // nb1),
        in_specs=[
            pl.BlockSpec((None, M, 3), lambda b, n: (b, 0, 0)),
            pl.BlockSpec((None, 3, nb1), lambda b, n: (b, 0, n)),
            pl.BlockSpec((None, C2, M), lambda b, n: (b, 0, 0)),
            pl.BlockSpec((None, C1, nb1), lambda b, n: (b, 0, n)),
            pl.BlockSpec((H1, C2 + C1), lambda b, n: (0, 0)),
        ],
        out_specs=[
            pl.BlockSpec((None, H1, nb1), lambda b, n: (b, 0, n)),
            pl.BlockSpec((H1, 128), lambda b, n: (0, 0)),
            pl.BlockSpec((H1, 128), lambda b, n: (0, 0)),
        ],
        out_shape=[
            jax.ShapeDtypeStruct((B, H1, N), jnp.float32),
            jax.ShapeDtypeStruct((H1, 128), jnp.float32),
            jax.ShapeDtypeStruct((H1, 128), jnp.float32),
        ],
    )(x2t, xyz1, points2, points1, W1)

    h2, s2, ss2 = pl.pallas_call(
        functools.partial(_k2, cnt=cnt),
        grid=(B, N // nb2),
        in_specs=[
            pl.BlockSpec((None, H1, nb2), lambda b, n: (b, 0, n)),
            pl.BlockSpec((H1, 128), lambda b, n: (0, 0)),
            pl.BlockSpec((H1, 128), lambda b, n: (0, 0)),
            pl.BlockSpec((H1, 1), lambda b, n: (0, 0)),
            pl.BlockSpec((H1, 1), lambda b, n: (0, 0)),
            pl.BlockSpec((H2, H1), lambda b, n: (0, 0)),
        ],
        out_specs=[
            pl.BlockSpec((None, H2, nb2), lambda b, n: (b, 0, n)),
            pl.BlockSpec((H2, 128), lambda b, n: (0, 0)),
            pl.BlockSpec((H2, 128), lambda b, n: (0, 0)),
        ],
        out_shape=[
            jax.ShapeDtypeStruct((B, H2, N), jnp.float32),
            jax.ShapeDtypeStruct((H2, 128), jnp.float32),
            jax.ShapeDtypeStruct((H2, 128), jnp.float32),
        ],
    )(h1, s1, ss1, g1[:, None], be1[:, None], W2)

    out = pl.pallas_call(
        functools.partial(_k3, cnt=cnt),
        grid=(B, N // nb3),
        in_specs=[
            pl.BlockSpec((None, H2, nb3), lambda b, n: (b, 0, n)),
            pl.BlockSpec((H2, 128), lambda b, n: (0, 0)),
            pl.BlockSpec((H2, 128), lambda b, n: (0, 0)),
            pl.BlockSpec((H2, 1), lambda b, n: (0, 0)),
            pl.BlockSpec((H2, 1), lambda b, n: (0, 0)),
        ],
        out_specs=pl.BlockSpec((None, H2, nb3), lambda b, n: (b, 0, n)),
        out_shape=jax.ShapeDtypeStruct((B, H2, N), jnp.float32),
    )(h2, s2, ss2, g2[:, None], be2[:, None])

    return out


# speedup vs baseline: 2.3512x; 1.0482x over previous
import functools

import jax
import jax.numpy as jnp
from jax.experimental import pallas as pl

_NB1 = 4096
_NB2 = 4096
_NB3 = 4096


def _k1(x2t_ref, x1_ref, p2_ref, p1_ref, w1_ref, h1_ref, s_ref, ss_ref):
    b = pl.program_id(0)
    nt = pl.program_id(1)
    M = x2t_ref.shape[0]
    C2 = p2_ref.shape[0]

    x2 = x2t_ref[...]
    x1 = x1_ref[...]
    n2 = jnp.sum(x2 * x2, axis=1, keepdims=True)
    n1 = jnp.sum(x1 * x1, axis=0, keepdims=True)
    x2a = jnp.concatenate([x2, n2, jnp.ones_like(n2)], axis=1)
    x1a = jnp.concatenate([-2.0 * x1, jnp.ones_like(n1), n1], axis=0)
    d = jax.lax.dot_general(
        x2a,
        x1a,
        (((1,), (0,)), ((), ())),
        precision=jax.lax.Precision.HIGHEST,
        preferred_element_type=jnp.float32,
    )

    inf = jnp.float32(jnp.inf)
    m1 = jnp.min(d, axis=0, keepdims=True)
    m2 = jnp.min(jnp.where(d > m1, d, inf), axis=0, keepdims=True)
    m3 = jnp.min(jnp.where(d > m2, d, inf), axis=0, keepdims=True)

    r1 = 1.0 / jnp.maximum(m1, 1e-10)
    r2 = 1.0 / jnp.maximum(m2, 1e-10)
    r3 = 1.0 / jnp.maximum(m3, 1e-10)
    rs = r1 + r2 + r3
    st = jnp.where(d <= m3, 1.0 / (jnp.maximum(d, 1e-10) * rs), 0.0)

    interp = jnp.dot(p2_ref[...], st, preferred_element_type=jnp.float32)
    h = jnp.dot(w1_ref[:, :C2], interp, preferred_element_type=jnp.float32)
    h = h + jnp.dot(w1_ref[:, C2:], p1_ref[...], preferred_element_type=jnp.float32)
    h1_ref[...] = h

    @pl.when(jnp.logical_and(b == 0, nt == 0))
    def _init():
        s_ref[...] = jnp.zeros_like(s_ref)
        ss_ref[...] = jnp.zeros_like(ss_ref)

    hh = h * h
    nb = h.shape[1]
    s_ref[...] += sum(h[:, j * 128 : (j + 1) * 128] for j in range(nb // 128))
    ss_ref[...] += sum(hh[:, j * 128 : (j + 1) * 128] for j in range(nb // 128))


def _bn_ac(s_ref, ss_ref, g_ref, be_ref, cnt):
    mean = jnp.sum(s_ref[...], axis=1, keepdims=True) * (1.0 / cnt)
    var = jnp.maximum(
        jnp.sum(ss_ref[...], axis=1, keepdims=True) * (1.0 / cnt) - mean * mean,
        0.0,
    )
    a = g_ref[...] * jax.lax.rsqrt(var + 1e-5)
    c = be_ref[...] - mean * a
    return a, c


def _k2(h1_ref, sin_ref, ssin_ref, g_ref, be_ref, w2_ref, h2_ref, s_ref, ss_ref, *, cnt):
    b = pl.program_id(0)
    nt = pl.program_id(1)
    a, c = _bn_ac(sin_ref, ssin_ref, g_ref, be_ref, cnt)
    hn = jnp.maximum(a * h1_ref[...] + c, 0.0)
    h2 = jnp.dot(w2_ref[...], hn, preferred_element_type=jnp.float32)
    h2_ref[...] = h2

    @pl.when(jnp.logical_and(b == 0, nt == 0))
    def _init():
        s_ref[...] = jnp.zeros_like(s_ref)
        ss_ref[...] = jnp.zeros_like(ss_ref)

    hh = h2 * h2
    nb = h2.shape[1]
    s_ref[...] += sum(h2[:, j * 128 : (j + 1) * 128] for j in range(nb // 128))
    ss_ref[...] += sum(hh[:, j * 128 : (j + 1) * 128] for j in range(nb // 128))


def _k3(h2_ref, sin_ref, ssin_ref, g_ref, be_ref, o_ref, *, cnt):
    a, c = _bn_ac(sin_ref, ssin_ref, g_ref, be_ref, cnt)
    o_ref[...] = jnp.maximum(a * h2_ref[...] + c, 0.0)


def kernel(xyz1, xyz2, points1, points2, W1, b1, g1, be1, W2, b2, g2, be2):
    B, _, N = xyz1.shape
    M = xyz2.shape[2]
    C1 = points1.shape[1]
    C2 = points2.shape[1]
    H1 = W1.shape[0]
    H2 = W2.shape[0]
    cnt = B * N
    nb1 = min(_NB1, N)
    nb2 = min(_NB2, N)
    nb3 = min(_NB3, N)

    x2t = jnp.transpose(xyz2, (0, 2, 1))

    h1, s1, ss1 = pl.pallas_call(
        _k1,
        grid=(B, N // nb1),
        in_specs=[
            pl.BlockSpec((None, M, 3), lambda b, n: (b, 0, 0)),
            pl.BlockSpec((None, 3, nb1), lambda b, n: (b, 0, n)),
            pl.BlockSpec((None, C2, M), lambda b, n: (b, 0, 0)),
            pl.BlockSpec((None, C1, nb1), lambda b, n: (b, 0, n)),
            pl.BlockSpec((H1, C2 + C1), lambda b, n: (0, 0)),
        ],
        out_specs=[
            pl.BlockSpec((None, H1, nb1), lambda b, n: (b, 0, n)),
            pl.BlockSpec((H1, 128), lambda b, n: (0, 0)),
            pl.BlockSpec((H1, 128), lambda b, n: (0, 0)),
        ],
        out_shape=[
            jax.ShapeDtypeStruct((B, H1, N), jnp.float32),
            jax.ShapeDtypeStruct((H1, 128), jnp.float32),
            jax.ShapeDtypeStruct((H1, 128), jnp.float32),
        ],
    )(x2t, xyz1, points2, points1, W1)

    h2, s2, ss2 = pl.pallas_call(
        functools.partial(_k2, cnt=cnt),
        grid=(B, N // nb2),
        in_specs=[
            pl.BlockSpec((None, H1, nb2), lambda b, n: (b, 0, n)),
            pl.BlockSpec((H1, 128), lambda b, n: (0, 0)),
            pl.BlockSpec((H1, 128), lambda b, n: (0, 0)),
            pl.BlockSpec((H1, 1), lambda b, n: (0, 0)),
            pl.BlockSpec((H1, 1), lambda b, n: (0, 0)),
            pl.BlockSpec((H2, H1), lambda b, n: (0, 0)),
        ],
        out_specs=[
            pl.BlockSpec((None, H2, nb2), lambda b, n: (b, 0, n)),
            pl.BlockSpec((H2, 128), lambda b, n: (0, 0)),
            pl.BlockSpec((H2, 128), lambda b, n: (0, 0)),
        ],
        out_shape=[
            jax.ShapeDtypeStruct((B, H2, N), jnp.float32),
            jax.ShapeDtypeStruct((H2, 128), jnp.float32),
            jax.ShapeDtypeStruct((H2, 128), jnp.float32),
        ],
    )(h1, s1, ss1, g1[:, None], be1[:, None], W2)

    out = pl.pallas_call(
        functools.partial(_k3, cnt=cnt),
        grid=(B, N // nb3),
        in_specs=[
            pl.BlockSpec((None, H2, nb3), lambda b, n: (b, 0, n)),
            pl.BlockSpec((H2, 128), lambda b, n: (0, 0)),
            pl.BlockSpec((H2, 128), lambda b, n: (0, 0)),
            pl.BlockSpec((H2, 1), lambda b, n: (0, 0)),
            pl.BlockSpec((H2, 1), lambda b, n: (0, 0)),
        ],
        out_specs=pl.BlockSpec((None, H2, nb3), lambda b, n: (b, 0, n)),
        out_shape=jax.ShapeDtypeStruct((B, H2, N), jnp.float32),
    )(h2, s2, ss2, g2[:, None], be2[:, None])

    return out
